# baseline (speedup 1.0000x reference)
import functools

import numpy as np

import jax
import jax.numpy as jnp
from jax import lax
from jax.experimental import pallas as pl
from jax.experimental.pallas import tpu as pltpu
from jax.experimental.pallas import tpu_sc as plsc

N = 10000
E = 320000
D = 128
R = 8
H = 8
HD = 16

TN = 400
NT = N // TN
CH = 64
NCH = E // CH
NW = 32
KMAX = -(-NCH // NW)
NP = 10240
ROWS_PER_TILE = NP // 16
H2 = H // 2

F32 = jnp.float32
HIGH = jax.lax.Precision.HIGHEST


def _lrelu(t):
    return jnp.maximum(t, 0.2 * t)


def _finalize_h(hacc_ref, s_ref, emh_ref, erep_ref):
    ha = hacc_ref[0] + hacc_ref[1]
    s8 = (s_ref[0] + s_ref[1]) * emh_ref[...]
    denom = jnp.dot(s8, erep_ref[...], precision=HIGH) + 1e-10
    numer = ha * jnp.dot(emh_ref[...], erep_ref[...], precision=HIGH)
    return _lrelu(numer / denom)


def _a_compute(hb, w_ref, al_ref, ar_ref, z_ref, el_ref, er_ref, gelp_ref):
    zb = jnp.dot(hb, w_ref[0], precision=HIGH)
    z_ref[0] = zb
    elb = jnp.dot(zb, al_ref[...], precision=HIGH)
    el_ref[0] = elb
    er_ref[0] = jnp.dot(zb, ar_ref[...], precision=HIGH)
    gelp_ref[0, 0] = jnp.max(elb, axis=0, keepdims=True)


def _a0_body(h_ref, w_ref, al_ref, ar_ref, z_ref, el_ref, er_ref, gelp_ref):
    _a_compute(h_ref[...], w_ref, al_ref, ar_ref, z_ref, el_ref, er_ref,
               gelp_ref)


def _af_body(hacc_ref, s_ref, emh_ref, erep_ref, w_ref, al_ref, ar_ref,
             z_ref, el_ref, er_ref, gelp_ref):
    hb = _finalize_h(hacc_ref, s_ref, emh_ref, erep_ref)
    _a_compute(hb, w_ref, al_ref, ar_ref, z_ref, el_ref, er_ref, gelp_ref)


_A_OUT_SPECS = [
    pl.BlockSpec((1, TN, D), lambda n, r: (r, n, 0)),
    pl.BlockSpec((1, TN, H), lambda n, r: (r, n, 0)),
    pl.BlockSpec((1, TN, H), lambda n, r: (r, n, 0)),
    pl.BlockSpec((1, 1, 1, H), lambda n, r: (n, r, 0, 0)),
]
_A_OUT_SHAPE = [
    jax.ShapeDtypeStruct((R, N, D), F32),
    jax.ShapeDtypeStruct((R, N, H), F32),
    jax.ShapeDtypeStruct((R, N, H), F32),
    jax.ShapeDtypeStruct((NT, R, 1, H), F32),
]
_A_W_SPECS = [
    pl.BlockSpec((1, D, D), lambda n, r: (r, 0, 0)),
    pl.BlockSpec((D, H), lambda n, r: (0, 0)),
    pl.BlockSpec((D, H), lambda n, r: (0, 0)),
]

_kernel_a0 = pl.pallas_call(
    _a0_body,
    grid=(NT, R),
    in_specs=[pl.BlockSpec((TN, D), lambda n, r: (n, 0))] + _A_W_SPECS,
    out_specs=_A_OUT_SPECS,
    out_shape=_A_OUT_SHAPE,
)

_kernel_af = pl.pallas_call(
    _af_body,
    grid=(NT, R),
    in_specs=[
        pl.BlockSpec((2, TN, D), lambda n, r: (0, n, 0)),
        pl.BlockSpec((2, TN, H), lambda n, r: (0, n, 0)),
        pl.BlockSpec((TN, H), lambda n, r: (n, 0)),
        pl.BlockSpec((H, D), lambda n, r: (0, 0)),
    ] + _A_W_SPECS,
    out_specs=_A_OUT_SPECS,
    out_shape=_A_OUT_SHAPE,
)


def _a2_body(gelp_ref, er_ref, emh_ref):
    gel = jnp.max(gelp_ref[...], axis=(0, 2))
    er = er_ref[...]
    t = _lrelu(gel[:, None, :] + er)
    emh_ref[...] = jnp.exp(-jnp.max(t, axis=0))


_kernel_a2 = pl.pallas_call(
    _a2_body,
    grid=(NT,),
    in_specs=[
        pl.BlockSpec((NT, R, 1, H), lambda n: (0, 0, 0, 0)),
        pl.BlockSpec((R, TN, H), lambda n: (0, n, 0)),
    ],
    out_specs=pl.BlockSpec((TN, H), lambda n: (n, 0)),
    out_shape=jax.ShapeDtypeStruct((N, H), F32),
)


_sc_mesh = plsc.VectorSubcoreMesh(core_axis_name="c", subcore_axis_name="s")

_NBUF = 2
_KU = -(-(KMAX + 2) // _NBUF)


def _sc_scratch():
    per_set = [
        pltpu.VMEM((CH,), jnp.int32),
        pltpu.VMEM((CH,), jnp.int32),
        pltpu.VMEM((CH,), jnp.int32),
        pltpu.VMEM((CH,), jnp.int32),
        pltpu.VMEM((H2, 2 * CH), jnp.int32),
        pltpu.VMEM((H2, 2 * CH), jnp.int32),
        pltpu.VMEM((H2, 2 * CH), jnp.int32),
        pltpu.VMEM((H2, 2 * CH), F32),
        pltpu.VMEM((H2, 2 * CH), F32),
        pltpu.VMEM((H2, 2 * CH), F32),
        pltpu.VMEM((CH, D), F32),
        pltpu.SemaphoreType.DMA,
        pltpu.SemaphoreType.DMA,
    ]
    return per_set * _NBUF + [
        pltpu.VMEM((CH * H,), F32),
        pltpu.VMEM_SHARED((NP * H,), F32),
        pltpu.VMEM_SHARED((NP, D), F32),
    ]


@functools.partial(
    pl.kernel,
    out_type=(
        jax.ShapeDtypeStruct((2, NP * H), F32),
        jax.ShapeDtypeStruct((2, NP, D), F32),
    ),
    mesh=_sc_mesh,
    scratch_types=_sc_scratch(),
)
def _edge_kernel(el_hbm, er_hbm, z_hbm, src_hbm, dst_hbm, et_hbm,
                 s_out, hacc_out, *scr):
    nper = 13
    sets = [scr[i * nper:(i + 1) * nper] for i in range(_NBUF)]
    zb1, s_sp, hacc_sp = scr[_NBUF * nper:]
    cid = lax.axis_index("c")
    sid = lax.axis_index("s")
    wid = sid * 2 + cid
    zeros16 = jnp.zeros((16,), F32)

    zg0 = sets[0][10]

    def _zero_zg(i, _):
        for j in range(D // 16):
            zg0[i, pl.ds(j * 16, 16)] = zeros16
        return 0
    lax.fori_loop(0, CH, _zero_zg, 0)

    def _zero_zb1(i, _):
        zb1[pl.ds(i * 16, 16)] = zeros16
        return 0
    lax.fori_loop(0, CH * H // 16, _zero_zb1, 0)

    row0 = sid * ROWS_PER_TILE
    for t in range(ROWS_PER_TILE // CH):
        pltpu.sync_copy(zg0, hacc_sp.at[pl.ds(row0 + t * CH, CH)])
        pltpu.sync_copy(zb1, s_sp.at[pl.ds((row0 + t * CH) * H, CH * H)])
    plsc.subcore_barrier()

    def fire_gathers(s, chunk_id):
        (srcb, dstb, etb, idxrs, ixs, ixd, ixm,
         elg, erg, wb, zg, gsem, ssem) = sets[s]
        base = chunk_id * CH
        pltpu.sync_copy(src_hbm.at[pl.ds(base, CH)], srcb)
        pltpu.sync_copy(dst_hbm.at[pl.ds(base, CH)], dstb)
        pltpu.sync_copy(et_hbm.at[pl.ds(base, CH)], etb)

        def _idx(i, _):
            sl = pl.ds(i * 16, 16)
            etN = etb[sl] * N
            rs = etN + srcb[sl]
            idxrs[sl] = rs
            rs8 = rs * H
            rd8 = (etN + dstb[sl]) * H
            dm8 = dstb[sl] * H
            for k in range(H2):
                sl0 = pl.ds(i * 16, 16)
                sl1 = pl.ds(CH + i * 16, 16)
                ixs[k, sl0] = rs8 + (2 * k)
                ixs[k, sl1] = rs8 + (2 * k + 1)
                ixd[k, sl0] = rd8 + (2 * k)
                ixd[k, sl1] = rd8 + (2 * k + 1)
                ixm[k, sl0] = dm8 + (2 * k)
                ixm[k, sl1] = dm8 + (2 * k + 1)
            return 0
        lax.fori_loop(0, CH // 16, _idx, 0)
        for src, dst in _gather_pairs(s):
            pltpu.async_copy(src, dst, gsem)

    def _gather_pairs(s):
        (srcb, dstb, etb, idxrs, ixs, ixd, ixm,
         elg, erg, wb, zg, gsem, ssem) = sets[s]
        pairs = [(z_hbm.at[idxrs], zg)]
        for k in range(H2):
            pairs.append((el_hbm.at[ixs.at[k]], elg.at[k]))
            pairs.append((er_hbm.at[ixd.at[k]], erg.at[k]))
        return pairs

    def drain_gathers(s):
        gsem = sets[s][11]
        for src, dst in _gather_pairs(s):
            pltpu.make_async_copy(src, dst, gsem).wait()

    def _scatter_pairs(s):
        (srcb, dstb, etb, idxrs, ixs, ixd, ixm,
         elg, erg, wb, zg, gsem, ssem) = sets[s]
        pairs = [(wb.at[k], s_sp.at[ixm.at[k]]) for k in range(H2)]
        pairs.append((zg, hacc_sp.at[dstb]))
        return pairs

    def fire_scatters(s):
        ssem = sets[s][12]
        for src, dst in _scatter_pairs(s):
            pltpu.async_copy(src, dst, ssem, add=True)

    def drain_scatters(s):
        ssem = sets[s][12]
        for src, dst in _scatter_pairs(s):
            pltpu.make_async_copy(src, dst, ssem).wait()

    def compute(s):
        (srcb, dstb, etb, idxrs, ixs, ixd, ixm,
         elg, erg, wb, zg, gsem, ssem) = sets[s]

        for k in range(H2):
            for v in range(2 * CH // 16):
                sl = pl.ds(v * 16, 16)
                t = elg[k, sl] + erg[k, sl]
                wb[k, sl] = jnp.exp(_lrelu(t))

        def _scale(cv, _):
            for h in range(H):
                wv = wb[h // 2, pl.ds((h % 2) * CH + cv * 16, 16)]
                sl = pl.ds(h * 16, 16)
                for t in range(16):
                    whc = wv.at[jnp.full((16,), t, jnp.int32)].get(
                        mode='promise_in_bounds')
                    c = cv * 16 + t
                    zg[c, sl] = zg[c, sl] * whc
            return 0
        lax.fori_loop(0, CH // 16, _scale, 0)

    fire_gathers(0, wid)

    def _group(k3, _):
        for j in range(_NBUF):
            i = _NBUF * k3 + j
            c_i = wid + NW * i
            c_ip1 = c_i + NW
            c_prev = wid + NW * (i + 1 - _NBUF)

            @pl.when((i + 1 - _NBUF >= 0) & (c_prev < NCH))
            def _():
                drain_scatters((j + 1) % _NBUF)

            @pl.when(c_ip1 < NCH)
            def _():
                fire_gathers((j + 1) % _NBUF, c_ip1)

            @pl.when(c_i < NCH)
            def _():
                drain_gathers(j)
                compute(j)
                fire_scatters(j)
        return 0
    lax.fori_loop(0, _KU, _group, 0)

    plsc.subcore_barrier()
    pltpu.sync_copy(s_sp.at[pl.ds(row0 * H, ROWS_PER_TILE * H)],
                    s_out.at[cid].at[pl.ds(row0 * H, ROWS_PER_TILE * H)])
    pltpu.sync_copy(hacc_sp.at[pl.ds(row0, ROWS_PER_TILE)],
                    hacc_out.at[cid].at[pl.ds(row0, ROWS_PER_TILE)])


def _f_body(hacc_ref, s_ref, emh_ref, erep_ref, x_ref, out_ref):
    out_ref[...] = _finalize_h(hacc_ref, s_ref, emh_ref, erep_ref) + x_ref[...]


_kernel_f_final = pl.pallas_call(
    _f_body,
    grid=(NT,),
    in_specs=[
        pl.BlockSpec((2, TN, D), lambda n: (0, n, 0)),
        pl.BlockSpec((2, TN, H), lambda n: (0, n, 0)),
        pl.BlockSpec((TN, H), lambda n: (n, 0)),
        pl.BlockSpec((H, D), lambda n: (0, 0)),
        pl.BlockSpec((TN, D), lambda n: (n, 0)),
    ],
    out_specs=pl.BlockSpec((TN, D), lambda n: (n, 0)),
    out_shape=jax.ShapeDtypeStruct((N, D), F32),
)


def _a_layout(a):
    idx = jnp.arange(D)
    head = idx // HD
    return jnp.zeros((D, H), F32).at[idx, head].set(a.reshape(-1))


def kernel(x, edge_index, edge_type, W0, al0, ar0, W1, al1, ar1,
           W2, al2, ar2, W3, al3, ar3):
    src = edge_index[0]
    dst = edge_index[1]
    et = edge_type
    params = [(W0, al0, ar0), (W1, al1, ar1), (W2, al2, ar2), (W3, al3, ar3)]

    erep = jnp.asarray(_EREP)
    out = None
    for l, (W, al, ar) in enumerate(params):
        if l == 0:
            z, el, er, gelp = _kernel_a0(x, W, _a_layout(al), _a_layout(ar))
        else:
            z, el, er, gelp = _kernel_af(hacc2, s2r, emh, erep,
                                         W, _a_layout(al), _a_layout(ar))
        emh = _kernel_a2(gelp, er)
        s2, hacc2 = _edge_kernel(el.reshape(-1), er.reshape(-1),
                                 z.reshape(R * N, D), src, dst, et)
        s2r = s2.reshape(2, NP, H)
    return _kernel_f_final(hacc2, s2r, emh, erep, x)


_EREP = np.repeat(np.eye(H, dtype=np.float32), HD, axis=1)


# speedup vs baseline: 1.0612x; 1.0612x over previous
import functools

import numpy as np

import jax
import jax.numpy as jnp
from jax import lax
from jax.experimental import pallas as pl
from jax.experimental.pallas import tpu as pltpu
from jax.experimental.pallas import tpu_sc as plsc

N = 10000
E = 320000
D = 128
R = 8
H = 8
HD = 16

TN = 400
NT = N // TN
CH = 64
NCH = E // CH
NW = 32
KMAX = -(-NCH // NW)
NP = 10240
ROWS_PER_TILE = NP // 16
H2 = H // 2

F32 = jnp.float32
HIGH = jax.lax.Precision.HIGHEST


def _lrelu(t):
    return jnp.maximum(t, 0.2 * t)


def _finalize_h(hacc_ref, s_ref, emh_ref, erep_ref):
    ha = hacc_ref[0] + hacc_ref[1]
    s8 = (s_ref[0] + s_ref[1]) * emh_ref[...]
    denom = jnp.dot(s8, erep_ref[...], precision=HIGH) + 1e-10
    numer = ha * jnp.dot(emh_ref[...], erep_ref[...], precision=HIGH)
    return _lrelu(numer / denom)


def _a_compute(hb, w_ref, al_ref, ar_ref, z_ref, el_ref, er_ref, gelp_ref):
    zb = jnp.dot(hb, w_ref[0], precision=HIGH)
    z_ref[0] = zb
    elb = jnp.dot(zb, al_ref[...], precision=HIGH)
    el_ref[0] = elb
    er_ref[0] = jnp.dot(zb, ar_ref[...], precision=HIGH)
    gelp_ref[0, 0] = jnp.max(elb, axis=0, keepdims=True)


def _a0_body(h_ref, w_ref, al_ref, ar_ref, z_ref, el_ref, er_ref, gelp_ref):
    _a_compute(h_ref[...], w_ref, al_ref, ar_ref, z_ref, el_ref, er_ref,
               gelp_ref)


def _af_body(hacc_ref, s_ref, emh_ref, erep_ref, w_ref, al_ref, ar_ref,
             z_ref, el_ref, er_ref, gelp_ref):
    hb = _finalize_h(hacc_ref, s_ref, emh_ref, erep_ref)
    _a_compute(hb, w_ref, al_ref, ar_ref, z_ref, el_ref, er_ref, gelp_ref)


_A_OUT_SPECS = [
    pl.BlockSpec((1, TN, D), lambda n, r: (r, n, 0)),
    pl.BlockSpec((1, TN, H), lambda n, r: (r, n, 0)),
    pl.BlockSpec((1, TN, H), lambda n, r: (r, n, 0)),
    pl.BlockSpec((1, 1, 1, H), lambda n, r: (n, r, 0, 0)),
]
_A_OUT_SHAPE = [
    jax.ShapeDtypeStruct((R, N, D), F32),
    jax.ShapeDtypeStruct((R, N, H), F32),
    jax.ShapeDtypeStruct((R, N, H), F32),
    jax.ShapeDtypeStruct((NT, R, 1, H), F32),
]
_A_W_SPECS = [
    pl.BlockSpec((1, D, D), lambda n, r: (r, 0, 0)),
    pl.BlockSpec((D, H), lambda n, r: (0, 0)),
    pl.BlockSpec((D, H), lambda n, r: (0, 0)),
]

_kernel_a0 = pl.pallas_call(
    _a0_body,
    grid=(NT, R),
    in_specs=[pl.BlockSpec((TN, D), lambda n, r: (n, 0))] + _A_W_SPECS,
    out_specs=_A_OUT_SPECS,
    out_shape=_A_OUT_SHAPE,
)

_kernel_af = pl.pallas_call(
    _af_body,
    grid=(NT, R),
    in_specs=[
        pl.BlockSpec((2, TN, D), lambda n, r: (0, n, 0)),
        pl.BlockSpec((2, TN, H), lambda n, r: (0, n, 0)),
        pl.BlockSpec((TN, H), lambda n, r: (n, 0)),
        pl.BlockSpec((H, D), lambda n, r: (0, 0)),
    ] + _A_W_SPECS,
    out_specs=_A_OUT_SPECS,
    out_shape=_A_OUT_SHAPE,
)


def _a2_body(gelp_ref, er_ref, emh_ref):
    gel = jnp.max(gelp_ref[...], axis=(0, 2))
    er = er_ref[...]
    t = _lrelu(gel[:, None, :] + er)
    emh_ref[...] = jnp.exp(-jnp.max(t, axis=0))


_kernel_a2 = pl.pallas_call(
    _a2_body,
    grid=(NT,),
    in_specs=[
        pl.BlockSpec((NT, R, 1, H), lambda n: (0, 0, 0, 0)),
        pl.BlockSpec((R, TN, H), lambda n: (0, n, 0)),
    ],
    out_specs=pl.BlockSpec((TN, H), lambda n: (n, 0)),
    out_shape=jax.ShapeDtypeStruct((N, H), F32),
)


_sc_mesh = plsc.VectorSubcoreMesh(core_axis_name="c", subcore_axis_name="s")

_NG = 2
_NS = 3
_NSLOT = 6
_KU = -(-(KMAX + 2) // _NSLOT)

_GSET = [
    pltpu.VMEM((CH,), jnp.int32),
    pltpu.VMEM((CH,), jnp.int32),
    pltpu.VMEM((CH,), jnp.int32),
    pltpu.VMEM((H2, 2 * CH), jnp.int32),
    pltpu.VMEM((H2, 2 * CH), jnp.int32),
    pltpu.VMEM((H2, 2 * CH), F32),
    pltpu.VMEM((H2, 2 * CH), F32),
    pltpu.SemaphoreType.DMA,
]
_SSET = [
    pltpu.VMEM((CH,), jnp.int32),
    pltpu.VMEM((H2, 2 * CH), jnp.int32),
    pltpu.VMEM((H2, 2 * CH), F32),
    pltpu.VMEM((CH, D), F32),
    pltpu.SemaphoreType.DMA,
]


def _sc_scratch():
    return _GSET * _NG + _SSET * _NS + [
        pltpu.VMEM((CH * H,), F32),
        pltpu.VMEM_SHARED((NP * H,), F32),
        pltpu.VMEM_SHARED((NP, D), F32),
    ]


@functools.partial(
    pl.kernel,
    out_type=(
        jax.ShapeDtypeStruct((2, NP * H), F32),
        jax.ShapeDtypeStruct((2, NP, D), F32),
    ),
    mesh=_sc_mesh,
    scratch_types=_sc_scratch(),
)
def _edge_kernel(el_hbm, er_hbm, z_hbm, src_hbm, dst_hbm, et_hbm,
                 s_out, hacc_out, *scr):
    ng, ns = len(_GSET), len(_SSET)
    gsets = [scr[g * ng:(g + 1) * ng] for g in range(_NG)]
    base = _NG * ng
    ssets = [scr[base + s * ns:base + (s + 1) * ns] for s in range(_NS)]
    zb1, s_sp, hacc_sp = scr[base + _NS * ns:]
    cid = lax.axis_index("c")
    sid = lax.axis_index("s")
    wid = sid * 2 + cid
    zeros16 = jnp.zeros((16,), F32)

    zg0 = ssets[0][3]

    def _zero_zg(i, _):
        for j in range(D // 16):
            zg0[i, pl.ds(j * 16, 16)] = zeros16
        return 0
    lax.fori_loop(0, CH, _zero_zg, 0)

    def _zero_zb1(i, _):
        zb1[pl.ds(i * 16, 16)] = zeros16
        return 0
    lax.fori_loop(0, CH * H // 16, _zero_zb1, 0)

    row0 = sid * ROWS_PER_TILE
    for t in range(ROWS_PER_TILE // CH):
        pltpu.sync_copy(zg0, hacc_sp.at[pl.ds(row0 + t * CH, CH)])
        pltpu.sync_copy(zb1, s_sp.at[pl.ds((row0 + t * CH) * H, CH * H)])
    plsc.subcore_barrier()

    def fire_gathers(g, s, chunk_id):
        srcb, etb, idxrs, ixs, ixd, elg, erg, gsem = gsets[g]
        dstb, ixm, wb, zg, ssem = ssets[s]
        base = chunk_id * CH
        pltpu.sync_copy(src_hbm.at[pl.ds(base, CH)], srcb)
        pltpu.sync_copy(dst_hbm.at[pl.ds(base, CH)], dstb)
        pltpu.sync_copy(et_hbm.at[pl.ds(base, CH)], etb)

        def _idx(i, _):
            sl = pl.ds(i * 16, 16)
            etN = etb[sl] * N
            rs = etN + srcb[sl]
            idxrs[sl] = rs
            rs8 = rs * H
            rd8 = (etN + dstb[sl]) * H
            dm8 = dstb[sl] * H
            for k in range(H2):
                sl0 = pl.ds(i * 16, 16)
                sl1 = pl.ds(CH + i * 16, 16)
                ixs[k, sl0] = rs8 + (2 * k)
                ixs[k, sl1] = rs8 + (2 * k + 1)
                ixd[k, sl0] = rd8 + (2 * k)
                ixd[k, sl1] = rd8 + (2 * k + 1)
                ixm[k, sl0] = dm8 + (2 * k)
                ixm[k, sl1] = dm8 + (2 * k + 1)
            return 0
        lax.fori_loop(0, CH // 16, _idx, 0)
        for src, dst in _gather_pairs(g, s):
            pltpu.async_copy(src, dst, gsem)

    def _gather_pairs(g, s):
        srcb, etb, idxrs, ixs, ixd, elg, erg, gsem = gsets[g]
        zg = ssets[s][3]
        pairs = [(z_hbm.at[idxrs], zg)]
        for k in range(H2):
            pairs.append((el_hbm.at[ixs.at[k]], elg.at[k]))
            pairs.append((er_hbm.at[ixd.at[k]], erg.at[k]))
        return pairs

    def drain_gathers(g, s):
        gsem = gsets[g][7]
        for src, dst in _gather_pairs(g, s):
            pltpu.make_async_copy(src, dst, gsem).wait()

    def _scatter_pairs(s):
        dstb, ixm, wb, zg, ssem = ssets[s]
        pairs = [(wb.at[k], s_sp.at[ixm.at[k]]) for k in range(H2)]
        pairs.append((zg, hacc_sp.at[dstb]))
        return pairs

    def fire_scatters(s):
        ssem = ssets[s][4]
        for src, dst in _scatter_pairs(s):
            pltpu.async_copy(src, dst, ssem, add=True)

    def drain_scatters(s):
        ssem = ssets[s][4]
        for src, dst in _scatter_pairs(s):
            pltpu.make_async_copy(src, dst, ssem).wait()

    def compute(g, s):
        srcb, etb, idxrs, ixs, ixd, elg, erg, gsem = gsets[g]
        dstb, ixm, wb, zg, ssem = ssets[s]

        for k in range(H2):
            for v in range(2 * CH // 16):
                sl = pl.ds(v * 16, 16)
                t = elg[k, sl] + erg[k, sl]
                wb[k, sl] = jnp.exp(_lrelu(t))

        def _scale(cv, _):
            for h in range(H):
                wv = wb[h // 2, pl.ds((h % 2) * CH + cv * 16, 16)]
                sl = pl.ds(h * 16, 16)
                for t in range(16):
                    whc = wv.at[jnp.full((16,), t, jnp.int32)].get(
                        mode='promise_in_bounds')
                    c = cv * 16 + t
                    zg[c, sl] = zg[c, sl] * whc
            return 0
        lax.fori_loop(0, CH // 16, _scale, 0)

    fire_gathers(0, 0, wid)

    def _group(k6, _):
        for j in range(_NSLOT):
            i = _NSLOT * k6 + j
            c_i = wid + NW * i
            c_ip1 = c_i + NW
            c_im2 = c_i - 2 * NW

            @pl.when((i >= 2) & (c_im2 < NCH))
            def _():
                drain_scatters((j + 1) % _NS)

            @pl.when(c_ip1 < NCH)
            def _():
                fire_gathers((j + 1) % _NG, (j + 1) % _NS, c_ip1)

            @pl.when(c_i < NCH)
            def _():
                drain_gathers(j % _NG, j % _NS)
                compute(j % _NG, j % _NS)
                fire_scatters(j % _NS)
        return 0
    lax.fori_loop(0, _KU, _group, 0)

    plsc.subcore_barrier()
    pltpu.sync_copy(s_sp.at[pl.ds(row0 * H, ROWS_PER_TILE * H)],
                    s_out.at[cid].at[pl.ds(row0 * H, ROWS_PER_TILE * H)])
    pltpu.sync_copy(hacc_sp.at[pl.ds(row0, ROWS_PER_TILE)],
                    hacc_out.at[cid].at[pl.ds(row0, ROWS_PER_TILE)])


def _f_body(hacc_ref, s_ref, emh_ref, erep_ref, x_ref, out_ref):
    out_ref[...] = _finalize_h(hacc_ref, s_ref, emh_ref, erep_ref) + x_ref[...]


_kernel_f_final = pl.pallas_call(
    _f_body,
    grid=(NT,),
    in_specs=[
        pl.BlockSpec((2, TN, D), lambda n: (0, n, 0)),
        pl.BlockSpec((2, TN, H), lambda n: (0, n, 0)),
        pl.BlockSpec((TN, H), lambda n: (n, 0)),
        pl.BlockSpec((H, D), lambda n: (0, 0)),
        pl.BlockSpec((TN, D), lambda n: (n, 0)),
    ],
    out_specs=pl.BlockSpec((TN, D), lambda n: (n, 0)),
    out_shape=jax.ShapeDtypeStruct((N, D), F32),
)


def _a_layout(a):
    idx = jnp.arange(D)
    head = idx // HD
    return jnp.zeros((D, H), F32).at[idx, head].set(a.reshape(-1))


def kernel(x, edge_index, edge_type, W0, al0, ar0, W1, al1, ar1,
           W2, al2, ar2, W3, al3, ar3):
    src = edge_index[0]
    dst = edge_index[1]
    et = edge_type
    params = [(W0, al0, ar0), (W1, al1, ar1), (W2, al2, ar2), (W3, al3, ar3)]

    erep = jnp.asarray(_EREP)
    out = None
    for l, (W, al, ar) in enumerate(params):
        if l == 0:
            z, el, er, gelp = _kernel_a0(x, W, _a_layout(al), _a_layout(ar))
        else:
            z, el, er, gelp = _kernel_af(hacc2, s2r, emh, erep,
                                         W, _a_layout(al), _a_layout(ar))
        emh = _kernel_a2(gelp, er)
        s2, hacc2 = _edge_kernel(el.reshape(-1), er.reshape(-1),
                                 z.reshape(R * N, D), src, dst, et)
        s2r = s2.reshape(2, NP, H)
    return _kernel_f_final(hacc2, s2r, emh, erep, x)


_EREP = np.repeat(np.eye(H, dtype=np.float32), HD, axis=1)


# speedup vs baseline: 1.1664x; 1.0991x over previous
import functools

import numpy as np

import jax
import jax.numpy as jnp
from jax import lax
from jax.experimental import pallas as pl
from jax.experimental.pallas import tpu as pltpu
from jax.experimental.pallas import tpu_sc as plsc

N = 10000
E = 320000
D = 128
R = 8
H = 8
HD = 16

TN = 400
NT = N // TN
CH = 64
NCH = E // CH
NW = 32
KMAX = -(-NCH // NW)
NP = 10240
ROWS_PER_TILE = NP // 16
H2 = H // 2

F32 = jnp.float32
HIGH = jax.lax.Precision.HIGHEST


def _lrelu(t):
    return jnp.maximum(t, 0.2 * t)


def _finalize_h(hacc_ref, s_ref, emh_ref, erep_ref):
    ha = hacc_ref[0] + hacc_ref[1]
    s8 = (s_ref[0] + s_ref[1]) * emh_ref[...]
    denom = jnp.dot(s8, erep_ref[...], precision=HIGH) + 1e-10
    numer = ha * jnp.dot(emh_ref[...], erep_ref[...], precision=HIGH)
    return _lrelu(numer / denom)


def _bf16x3_dot(a, b):
    bf = jnp.bfloat16
    a_hi = a.astype(bf)
    a_lo = (a - a_hi.astype(F32)).astype(bf)
    b_hi = b.astype(bf)
    b_lo = (b - b_hi.astype(F32)).astype(bf)
    d = functools.partial(jnp.dot, preferred_element_type=F32)
    return d(a_hi, b_hi) + (d(a_hi, b_lo) + d(a_lo, b_hi))


def _a_compute(hb, w_ref, al_ref, ar_ref, z_ref, el_ref, er_ref, gelp_ref):
    zb = _bf16x3_dot(hb, w_ref[0])
    z_ref[0] = zb
    elb = jnp.dot(zb, al_ref[...], precision=HIGH)
    el_ref[0] = elb
    er_ref[0] = jnp.dot(zb, ar_ref[...], precision=HIGH)
    gelp_ref[0, 0] = jnp.max(elb, axis=0, keepdims=True)


def _a0_body(h_ref, w_ref, al_ref, ar_ref, z_ref, el_ref, er_ref, gelp_ref):
    _a_compute(h_ref[...], w_ref, al_ref, ar_ref, z_ref, el_ref, er_ref,
               gelp_ref)


def _af_body(hacc_ref, s_ref, emh_ref, erep_ref, w_ref, al_ref, ar_ref,
             z_ref, el_ref, er_ref, gelp_ref, hb_ref):
    @pl.when(pl.program_id(1) == 0)
    def _():
        hb_ref[...] = _finalize_h(hacc_ref, s_ref, emh_ref, erep_ref)
    _a_compute(hb_ref[...], w_ref, al_ref, ar_ref, z_ref, el_ref, er_ref,
               gelp_ref)


_A_OUT_SPECS = [
    pl.BlockSpec((1, TN, D), lambda n, r: (r, n, 0)),
    pl.BlockSpec((1, TN, H), lambda n, r: (r, n, 0)),
    pl.BlockSpec((1, TN, H), lambda n, r: (r, n, 0)),
    pl.BlockSpec((1, 1, 1, H), lambda n, r: (n, r, 0, 0)),
]
_A_OUT_SHAPE = [
    jax.ShapeDtypeStruct((R, N, D), F32),
    jax.ShapeDtypeStruct((R, N, H), F32),
    jax.ShapeDtypeStruct((R, N, H), F32),
    jax.ShapeDtypeStruct((NT, R, 1, H), F32),
]
_A_W_SPECS = [
    pl.BlockSpec((1, D, D), lambda n, r: (r, 0, 0)),
    pl.BlockSpec((D, H), lambda n, r: (0, 0)),
    pl.BlockSpec((D, H), lambda n, r: (0, 0)),
]

_kernel_a0 = pl.pallas_call(
    _a0_body,
    grid=(NT, R),
    in_specs=[pl.BlockSpec((TN, D), lambda n, r: (n, 0))] + _A_W_SPECS,
    out_specs=_A_OUT_SPECS,
    out_shape=_A_OUT_SHAPE,
)

_kernel_af = pl.pallas_call(
    _af_body,
    grid=(NT, R),
    in_specs=[
        pl.BlockSpec((2, TN, D), lambda n, r: (0, n, 0)),
        pl.BlockSpec((2, TN, H), lambda n, r: (0, n, 0)),
        pl.BlockSpec((TN, H), lambda n, r: (n, 0)),
        pl.BlockSpec((H, D), lambda n, r: (0, 0)),
    ] + _A_W_SPECS,
    out_specs=_A_OUT_SPECS,
    out_shape=_A_OUT_SHAPE,
    scratch_shapes=[pltpu.VMEM((TN, D), F32)],
)


def _a2_body(gelp_ref, er_ref, emh_ref):
    gel = jnp.max(gelp_ref[...], axis=(0, 2))
    er = er_ref[...]
    t = _lrelu(gel[:, None, :] + er)
    emh_ref[...] = jnp.exp(-jnp.max(t, axis=0))


_kernel_a2 = pl.pallas_call(
    _a2_body,
    grid=(NT,),
    in_specs=[
        pl.BlockSpec((NT, R, 1, H), lambda n: (0, 0, 0, 0)),
        pl.BlockSpec((R, TN, H), lambda n: (0, n, 0)),
    ],
    out_specs=pl.BlockSpec((TN, H), lambda n: (n, 0)),
    out_shape=jax.ShapeDtypeStruct((N, H), F32),
)


_sc_mesh = plsc.VectorSubcoreMesh(core_axis_name="c", subcore_axis_name="s")

_NG = 2
_NS = 3
_NSLOT = 6
_KU = -(-(KMAX + 2) // _NSLOT)

_GSET = [
    pltpu.VMEM((CH,), jnp.int32),
    pltpu.VMEM((CH,), jnp.int32),
    pltpu.VMEM((CH,), jnp.int32),
    pltpu.VMEM((H2, 2 * CH), jnp.int32),
    pltpu.VMEM((H2, 2 * CH), jnp.int32),
    pltpu.VMEM((H2, 2 * CH), F32),
    pltpu.VMEM((H2, 2 * CH), F32),
    pltpu.SemaphoreType.DMA,
]
_SSET = [
    pltpu.VMEM((CH,), jnp.int32),
    pltpu.VMEM((H2, 2 * CH), jnp.int32),
    pltpu.VMEM((H2, 2 * CH), F32),
    pltpu.VMEM((CH, D), F32),
    pltpu.SemaphoreType.DMA,
]


def _sc_scratch():
    return _GSET * _NG + _SSET * _NS + [
        pltpu.VMEM((CH * H,), F32),
        pltpu.VMEM_SHARED((NP * H,), F32),
        pltpu.VMEM_SHARED((NP, D), F32),
    ]


@functools.partial(
    pl.kernel,
    out_type=(
        jax.ShapeDtypeStruct((2, NP * H), F32),
        jax.ShapeDtypeStruct((2, NP, D), F32),
    ),
    mesh=_sc_mesh,
    scratch_types=_sc_scratch(),
)
def _edge_kernel(el_hbm, er_hbm, z_hbm, src_hbm, dst_hbm, et_hbm,
                 s_out, hacc_out, *scr):
    ng, ns = len(_GSET), len(_SSET)
    gsets = [scr[g * ng:(g + 1) * ng] for g in range(_NG)]
    base = _NG * ng
    ssets = [scr[base + s * ns:base + (s + 1) * ns] for s in range(_NS)]
    zb1, s_sp, hacc_sp = scr[base + _NS * ns:]
    cid = lax.axis_index("c")
    sid = lax.axis_index("s")
    wid = sid * 2 + cid
    zeros16 = jnp.zeros((16,), F32)

    zg0 = ssets[0][3]

    def _zero_zg(i, _):
        for j in range(D // 16):
            zg0[i, pl.ds(j * 16, 16)] = zeros16
        return 0
    lax.fori_loop(0, CH, _zero_zg, 0)

    def _zero_zb1(i, _):
        zb1[pl.ds(i * 16, 16)] = zeros16
        return 0
    lax.fori_loop(0, CH * H // 16, _zero_zb1, 0)

    row0 = sid * ROWS_PER_TILE
    for t in range(ROWS_PER_TILE // CH):
        pltpu.sync_copy(zg0, hacc_sp.at[pl.ds(row0 + t * CH, CH)])
        pltpu.sync_copy(zb1, s_sp.at[pl.ds((row0 + t * CH) * H, CH * H)])
    plsc.subcore_barrier()

    def fire_gathers(g, s, chunk_id):
        srcb, etb, idxrs, ixs, ixd, elg, erg, gsem = gsets[g]
        dstb, ixm, wb, zg, ssem = ssets[s]
        base = chunk_id * CH
        pltpu.sync_copy(src_hbm.at[pl.ds(base, CH)], srcb)
        pltpu.sync_copy(dst_hbm.at[pl.ds(base, CH)], dstb)
        pltpu.sync_copy(et_hbm.at[pl.ds(base, CH)], etb)

        def _idx(i, _):
            sl = pl.ds(i * 16, 16)
            etN = etb[sl] * N
            rs = etN + srcb[sl]
            idxrs[sl] = rs
            rs8 = rs * H
            rd8 = (etN + dstb[sl]) * H
            dm8 = dstb[sl] * H
            for k in range(H2):
                sl0 = pl.ds(i * 16, 16)
                sl1 = pl.ds(CH + i * 16, 16)
                ixs[k, sl0] = rs8 + (2 * k)
                ixs[k, sl1] = rs8 + (2 * k + 1)
                ixd[k, sl0] = rd8 + (2 * k)
                ixd[k, sl1] = rd8 + (2 * k + 1)
                ixm[k, sl0] = dm8 + (2 * k)
                ixm[k, sl1] = dm8 + (2 * k + 1)
            return 0
        lax.fori_loop(0, CH // 16, _idx, 0)
        for src, dst in _gather_pairs(g, s):
            pltpu.async_copy(src, dst, gsem)

    def _gather_pairs(g, s):
        srcb, etb, idxrs, ixs, ixd, elg, erg, gsem = gsets[g]
        zg = ssets[s][3]
        pairs = [(z_hbm.at[idxrs], zg)]
        for k in range(H2):
            pairs.append((el_hbm.at[ixs.at[k]], elg.at[k]))
            pairs.append((er_hbm.at[ixd.at[k]], erg.at[k]))
        return pairs

    def drain_gathers(g, s):
        gsem = gsets[g][7]
        for src, dst in _gather_pairs(g, s):
            pltpu.make_async_copy(src, dst, gsem).wait()

    def _scatter_pairs(s):
        dstb, ixm, wb, zg, ssem = ssets[s]
        pairs = [(wb.at[k], s_sp.at[ixm.at[k]]) for k in range(H2)]
        pairs.append((zg, hacc_sp.at[dstb]))
        return pairs

    def fire_scatters(s):
        ssem = ssets[s][4]
        for src, dst in _scatter_pairs(s):
            pltpu.async_copy(src, dst, ssem, add=True)

    def drain_scatters(s):
        ssem = ssets[s][4]
        for src, dst in _scatter_pairs(s):
            pltpu.make_async_copy(src, dst, ssem).wait()

    def compute(g, s):
        srcb, etb, idxrs, ixs, ixd, elg, erg, gsem = gsets[g]
        dstb, ixm, wb, zg, ssem = ssets[s]

        for k in range(H2):
            for v in range(2 * CH // 16):
                sl = pl.ds(v * 16, 16)
                t = elg[k, sl] + erg[k, sl]
                wb[k, sl] = jnp.exp(_lrelu(t))

        def _scale(cv, _):
            for h in range(H):
                wv = wb[h // 2, pl.ds((h % 2) * CH + cv * 16, 16)]
                sl = pl.ds(h * 16, 16)
                for t in range(16):
                    whc = wv.at[jnp.full((16,), t, jnp.int32)].get(
                        mode='promise_in_bounds')
                    c = cv * 16 + t
                    zg[c, sl] = zg[c, sl] * whc
            return 0
        lax.fori_loop(0, CH // 16, _scale, 0)

    fire_gathers(0, 0, wid)

    def _group(k6, _):
        for j in range(_NSLOT):
            i = _NSLOT * k6 + j
            c_i = wid + NW * i
            c_ip1 = c_i + NW
            c_im2 = c_i - 2 * NW

            @pl.when((i >= 2) & (c_im2 < NCH))
            def _():
                drain_scatters((j + 1) % _NS)

            @pl.when(c_ip1 < NCH)
            def _():
                fire_gathers((j + 1) % _NG, (j + 1) % _NS, c_ip1)

            @pl.when(c_i < NCH)
            def _():
                drain_gathers(j % _NG, j % _NS)
                compute(j % _NG, j % _NS)
                fire_scatters(j % _NS)
        return 0
    lax.fori_loop(0, _KU, _group, 0)

    plsc.subcore_barrier()
    pltpu.sync_copy(s_sp.at[pl.ds(row0 * H, ROWS_PER_TILE * H)],
                    s_out.at[cid].at[pl.ds(row0 * H, ROWS_PER_TILE * H)])
    pltpu.sync_copy(hacc_sp.at[pl.ds(row0, ROWS_PER_TILE)],
                    hacc_out.at[cid].at[pl.ds(row0, ROWS_PER_TILE)])


def _f_body(hacc_ref, s_ref, emh_ref, erep_ref, x_ref, out_ref):
    out_ref[...] = _finalize_h(hacc_ref, s_ref, emh_ref, erep_ref) + x_ref[...]


_kernel_f_final = pl.pallas_call(
    _f_body,
    grid=(NT,),
    in_specs=[
        pl.BlockSpec((2, TN, D), lambda n: (0, n, 0)),
        pl.BlockSpec((2, TN, H), lambda n: (0, n, 0)),
        pl.BlockSpec((TN, H), lambda n: (n, 0)),
        pl.BlockSpec((H, D), lambda n: (0, 0)),
        pl.BlockSpec((TN, D), lambda n: (n, 0)),
    ],
    out_specs=pl.BlockSpec((TN, D), lambda n: (n, 0)),
    out_shape=jax.ShapeDtypeStruct((N, D), F32),
)


def _a_layout(a):
    idx = jnp.arange(D)
    head = idx // HD
    return jnp.zeros((D, H), F32).at[idx, head].set(a.reshape(-1))


def kernel(x, edge_index, edge_type, W0, al0, ar0, W1, al1, ar1,
           W2, al2, ar2, W3, al3, ar3):
    src = edge_index[0]
    dst = edge_index[1]
    et = edge_type
    params = [(W0, al0, ar0), (W1, al1, ar1), (W2, al2, ar2), (W3, al3, ar3)]

    erep = jnp.asarray(_EREP)
    out = None
    for l, (W, al, ar) in enumerate(params):
        if l == 0:
            z, el, er, gelp = _kernel_a0(x, W, _a_layout(al), _a_layout(ar))
        else:
            z, el, er, gelp = _kernel_af(hacc2, s2r, emh, erep,
                                         W, _a_layout(al), _a_layout(ar))
        emh = _kernel_a2(gelp, er)
        s2, hacc2 = _edge_kernel(el.reshape(-1), er.reshape(-1),
                                 z.reshape(R * N, D), src, dst, et)
        s2r = s2.reshape(2, NP, H)
    return _kernel_f_final(hacc2, s2r, emh, erep, x)


_EREP = np.repeat(np.eye(H, dtype=np.float32), HD, axis=1)


# speedup vs baseline: 1.2241x; 1.0495x over previous
import functools

import numpy as np

import jax
import jax.numpy as jnp
from jax import lax
from jax.experimental import pallas as pl
from jax.experimental.pallas import tpu as pltpu
from jax.experimental.pallas import tpu_sc as plsc

N = 10000
E = 320000
D = 128
R = 8
H = 8
HD = 16

TN = 400
NT = N // TN
CH = 64
NCH = E // CH
NW = 32
KMAX = -(-NCH // NW)
NP = 10240
ROWS_PER_TILE = NP // 16
H2 = H // 2

F32 = jnp.float32
HIGH = jax.lax.Precision.HIGHEST


def _lrelu(t):
    return jnp.maximum(t, 0.2 * t)


def _finalize_h(hacc_ref, s_ref, emh_ref, erep_ref):
    ha = hacc_ref[0] + hacc_ref[1]
    s8 = (s_ref[0] + s_ref[1]) * emh_ref[...]
    denom = jnp.dot(s8, erep_ref[...], precision=HIGH) + 1e-10
    numer = ha * jnp.dot(emh_ref[...], erep_ref[...], precision=HIGH)
    return _lrelu(numer / denom)


def _bf16x3_dot(a, b):
    bf = jnp.bfloat16
    a_hi = a.astype(bf)
    a_lo = (a - a_hi.astype(F32)).astype(bf)
    b_hi = b.astype(bf)
    b_lo = (b - b_hi.astype(F32)).astype(bf)
    d = functools.partial(jnp.dot, preferred_element_type=F32)
    return d(a_hi, b_hi) + (d(a_hi, b_lo) + d(a_lo, b_hi))


def _a_compute(hb, wa_ref, z_ref, el_ref, er_ref, gelp_ref):
    t = _bf16x3_dot(hb, wa_ref[0])
    z_ref[0] = t[:, :D]
    elb = t[:, D:D + H]
    el_ref[0] = elb
    er_ref[0] = t[:, D + H:]
    gelp_ref[0, 0] = jnp.max(elb, axis=0, keepdims=True)


def _a0_body(h_ref, wa_ref, z_ref, el_ref, er_ref, gelp_ref):
    _a_compute(h_ref[...], wa_ref, z_ref, el_ref, er_ref, gelp_ref)


def _af_body(hacc_ref, s_ref, emh_ref, erep_ref, wa_ref,
             z_ref, el_ref, er_ref, gelp_ref, hb_ref):
    @pl.when(pl.program_id(1) == 0)
    def _():
        hb_ref[...] = _finalize_h(hacc_ref, s_ref, emh_ref, erep_ref)
    _a_compute(hb_ref[...], wa_ref, z_ref, el_ref, er_ref, gelp_ref)


NH128 = N * H // 128
TNH128 = TN * H // 128

_A_OUT_SPECS = [
    pl.BlockSpec((1, TN, D), lambda n, r: (r, n, 0)),
    pl.BlockSpec((1, TN, H), lambda n, r: (r, n, 0)),
    pl.BlockSpec((1, TN, H), lambda n, r: (r, n, 0)),
    pl.BlockSpec((1, 1, 1, H), lambda n, r: (n, r, 0, 0)),
]
_A_OUT_SHAPE = [
    jax.ShapeDtypeStruct((R, N, D), F32),
    jax.ShapeDtypeStruct((R, N, H), F32),
    jax.ShapeDtypeStruct((R, N, H), F32),
    jax.ShapeDtypeStruct((NT, R, 1, H), F32),
]
_A_W_SPECS = [
    pl.BlockSpec((1, D, D + 2 * H), lambda n, r: (r, 0, 0)),
]

_kernel_a0 = pl.pallas_call(
    _a0_body,
    grid=(NT, R),
    in_specs=[pl.BlockSpec((TN, D), lambda n, r: (n, 0))] + _A_W_SPECS,
    out_specs=_A_OUT_SPECS,
    out_shape=_A_OUT_SHAPE,
)

_kernel_af = pl.pallas_call(
    _af_body,
    grid=(NT, R),
    in_specs=[
        pl.BlockSpec((2, TN, D), lambda n, r: (0, n, 0)),
        pl.BlockSpec((2, TN, H), lambda n, r: (0, n, 0)),
        pl.BlockSpec((TN, H), lambda n, r: (n, 0)),
        pl.BlockSpec((H, D), lambda n, r: (0, 0)),
    ] + _A_W_SPECS,
    out_specs=_A_OUT_SPECS,
    out_shape=_A_OUT_SHAPE,
    scratch_shapes=[pltpu.VMEM((TN, D), F32)],
)


def _a2_body(gelp_ref, er_ref, emh_ref):
    gel = jnp.max(gelp_ref[...], axis=(0, 2))
    er = er_ref[...]
    t = _lrelu(gel[:, None, :] + er)
    emh_ref[...] = jnp.exp(-jnp.max(t, axis=0))


_kernel_a2 = pl.pallas_call(
    _a2_body,
    grid=(NT,),
    in_specs=[
        pl.BlockSpec((NT, R, 1, H), lambda n: (0, 0, 0, 0)),
        pl.BlockSpec((R, TN, H), lambda n: (0, n, 0)),
    ],
    out_specs=pl.BlockSpec((TN, H), lambda n: (n, 0)),
    out_shape=jax.ShapeDtypeStruct((N, H), F32),
)


_sc_mesh = plsc.VectorSubcoreMesh(core_axis_name="c", subcore_axis_name="s")

_NG = 2
_NS = 3
_NSLOT = 6
_KU = -(-(KMAX + 2) // _NSLOT)

_GSET = [
    pltpu.VMEM((CH,), jnp.int32),
    pltpu.VMEM((CH,), jnp.int32),
    pltpu.VMEM((CH,), jnp.int32),
    pltpu.VMEM((H2, 2 * CH), jnp.int32),
    pltpu.VMEM((H2, 2 * CH), jnp.int32),
    pltpu.VMEM((H2, 2 * CH), F32),
    pltpu.VMEM((H2, 2 * CH), F32),
    pltpu.SemaphoreType.DMA,
]
_SSET = [
    pltpu.VMEM((CH,), jnp.int32),
    pltpu.VMEM((H2, 2 * CH), jnp.int32),
    pltpu.VMEM((H2, 2 * CH), F32),
    pltpu.VMEM((CH, D), F32),
    pltpu.SemaphoreType.DMA,
]


def _sc_scratch():
    return _GSET * _NG + _SSET * _NS + [
        pltpu.VMEM((CH * H,), F32),
        pltpu.VMEM_SHARED((NP * H,), F32),
        pltpu.VMEM_SHARED((NP, D), F32),
    ]


@functools.partial(
    pl.kernel,
    out_type=(
        jax.ShapeDtypeStruct((2, NP * H), F32),
        jax.ShapeDtypeStruct((2, NP, D), F32),
    ),
    mesh=_sc_mesh,
    scratch_types=_sc_scratch(),
)
def _edge_kernel(el_hbm, er_hbm, z_hbm, src_hbm, dst_hbm, et_hbm,
                 s_out, hacc_out, *scr):
    ng, ns = len(_GSET), len(_SSET)
    gsets = [scr[g * ng:(g + 1) * ng] for g in range(_NG)]
    base = _NG * ng
    ssets = [scr[base + s * ns:base + (s + 1) * ns] for s in range(_NS)]
    zb1, s_sp, hacc_sp = scr[base + _NS * ns:]
    cid = lax.axis_index("c")
    sid = lax.axis_index("s")
    wid = sid * 2 + cid
    zeros16 = jnp.zeros((16,), F32)

    zg0 = ssets[0][3]

    def _zero_zg(i, _):
        for j in range(D // 16):
            zg0[i, pl.ds(j * 16, 16)] = zeros16
        return 0
    lax.fori_loop(0, CH, _zero_zg, 0)

    def _zero_zb1(i, _):
        zb1[pl.ds(i * 16, 16)] = zeros16
        return 0
    lax.fori_loop(0, CH * H // 16, _zero_zb1, 0)

    row0 = sid * ROWS_PER_TILE
    for t in range(ROWS_PER_TILE // CH):
        pltpu.sync_copy(zg0, hacc_sp.at[pl.ds(row0 + t * CH, CH)])
        pltpu.sync_copy(zb1, s_sp.at[pl.ds((row0 + t * CH) * H, CH * H)])
    plsc.subcore_barrier()

    def fire_gathers(g, s, chunk_id):
        srcb, etb, idxrs, ixs, ixd, elg, erg, gsem = gsets[g]
        dstb, ixm, wb, zg, ssem = ssets[s]
        base = chunk_id * CH
        pltpu.sync_copy(src_hbm.at[pl.ds(base, CH)], srcb)
        pltpu.sync_copy(dst_hbm.at[pl.ds(base, CH)], dstb)
        pltpu.sync_copy(et_hbm.at[pl.ds(base, CH)], etb)

        def _idx(i, _):
            sl = pl.ds(i * 16, 16)
            etN = etb[sl] * N
            rs = etN + srcb[sl]
            idxrs[sl] = rs
            rs8 = rs * H
            rd8 = (etN + dstb[sl]) * H
            dm8 = dstb[sl] * H
            for k in range(H2):
                sl0 = pl.ds(i * 16, 16)
                sl1 = pl.ds(CH + i * 16, 16)
                ixs[k, sl0] = rs8 + (2 * k)
                ixs[k, sl1] = rs8 + (2 * k + 1)
                ixd[k, sl0] = rd8 + (2 * k)
                ixd[k, sl1] = rd8 + (2 * k + 1)
                ixm[k, sl0] = dm8 + (2 * k)
                ixm[k, sl1] = dm8 + (2 * k + 1)
            return 0
        lax.fori_loop(0, CH // 16, _idx, 0)
        for src, dst in _gather_pairs(g, s):
            pltpu.async_copy(src, dst, gsem)

    def _gather_pairs(g, s):
        srcb, etb, idxrs, ixs, ixd, elg, erg, gsem = gsets[g]
        zg = ssets[s][3]
        pairs = [(z_hbm.at[idxrs], zg)]
        for k in range(H2):
            pairs.append((el_hbm.at[ixs.at[k]], elg.at[k]))
            pairs.append((er_hbm.at[ixd.at[k]], erg.at[k]))
        return pairs

    def drain_gathers(g, s):
        gsem = gsets[g][7]
        for src, dst in _gather_pairs(g, s):
            pltpu.make_async_copy(src, dst, gsem).wait()

    def _scatter_pairs(s):
        dstb, ixm, wb, zg, ssem = ssets[s]
        pairs = [(wb.at[k], s_sp.at[ixm.at[k]]) for k in range(H2)]
        pairs.append((zg, hacc_sp.at[dstb]))
        return pairs

    def fire_scatters(s):
        ssem = ssets[s][4]
        for src, dst in _scatter_pairs(s):
            pltpu.async_copy(src, dst, ssem, add=True)

    def drain_scatters(s):
        ssem = ssets[s][4]
        for src, dst in _scatter_pairs(s):
            pltpu.make_async_copy(src, dst, ssem).wait()

    def compute(g, s):
        srcb, etb, idxrs, ixs, ixd, elg, erg, gsem = gsets[g]
        dstb, ixm, wb, zg, ssem = ssets[s]

        for k in range(H2):
            for v in range(2 * CH // 16):
                sl = pl.ds(v * 16, 16)
                t = elg[k, sl] + erg[k, sl]
                wb[k, sl] = jnp.exp(_lrelu(t))

        def _scale(cv, _):
            for h in range(H):
                wv = wb[h // 2, pl.ds((h % 2) * CH + cv * 16, 16)]
                sl = pl.ds(h * 16, 16)
                for t in range(16):
                    whc = wv.at[jnp.full((16,), t, jnp.int32)].get(
                        mode='promise_in_bounds')
                    c = cv * 16 + t
                    zg[c, sl] = zg[c, sl] * whc
            return 0
        lax.fori_loop(0, CH // 16, _scale, 0)

    fire_gathers(0, 0, wid)

    def _group(k6, _):
        for j in range(_NSLOT):
            i = _NSLOT * k6 + j
            c_i = wid + NW * i
            c_ip1 = c_i + NW
            c_im2 = c_i - 2 * NW

            @pl.when((i >= 2) & (c_im2 < NCH))
            def _():
                drain_scatters((j + 1) % _NS)

            @pl.when(c_ip1 < NCH)
            def _():
                fire_gathers((j + 1) % _NG, (j + 1) % _NS, c_ip1)

            @pl.when(c_i < NCH)
            def _():
                drain_gathers(j % _NG, j % _NS)
                compute(j % _NG, j % _NS)
                fire_scatters(j % _NS)
        return 0
    lax.fori_loop(0, _KU, _group, 0)

    plsc.subcore_barrier()
    pltpu.sync_copy(s_sp.at[pl.ds(row0 * H, ROWS_PER_TILE * H)],
                    s_out.at[cid].at[pl.ds(row0 * H, ROWS_PER_TILE * H)])
    pltpu.sync_copy(hacc_sp.at[pl.ds(row0, ROWS_PER_TILE)],
                    hacc_out.at[cid].at[pl.ds(row0, ROWS_PER_TILE)])


def _f_body(hacc_ref, s_ref, emh_ref, erep_ref, x_ref, out_ref):
    out_ref[...] = _finalize_h(hacc_ref, s_ref, emh_ref, erep_ref) + x_ref[...]


_kernel_f_final = pl.pallas_call(
    _f_body,
    grid=(NT,),
    in_specs=[
        pl.BlockSpec((2, TN, D), lambda n: (0, n, 0)),
        pl.BlockSpec((2, TN, H), lambda n: (0, n, 0)),
        pl.BlockSpec((TN, H), lambda n: (n, 0)),
        pl.BlockSpec((H, D), lambda n: (0, 0)),
        pl.BlockSpec((TN, D), lambda n: (n, 0)),
    ],
    out_specs=pl.BlockSpec((TN, D), lambda n: (n, 0)),
    out_shape=jax.ShapeDtypeStruct((N, D), F32),
)


def _a_layout(a):
    idx = jnp.arange(D)
    head = idx // HD
    return jnp.zeros((D, H), F32).at[idx, head].set(a.reshape(-1))


def kernel(x, edge_index, edge_type, W0, al0, ar0, W1, al1, ar1,
           W2, al2, ar2, W3, al3, ar3):
    src = edge_index[0]
    dst = edge_index[1]
    et = edge_type
    params = [(W0, al0, ar0), (W1, al1, ar1), (W2, al2, ar2), (W3, al3, ar3)]

    erep = jnp.asarray(_EREP)
    out = None
    for l, (W, al, ar) in enumerate(params):
        WA = jnp.concatenate(
            [W, jnp.matmul(W, _a_layout(al), precision=HIGH),
             jnp.matmul(W, _a_layout(ar), precision=HIGH)], axis=-1)
        if l == 0:
            z, el, er, gelp = _kernel_a0(x, WA)
        else:
            z, el, er, gelp = _kernel_af(hacc2, s2r, emh, erep, WA)
        emh = _kernel_a2(gelp, er)
        s2, hacc2 = _edge_kernel(el.reshape(-1), er.reshape(-1),
                                 z.reshape(R * N, D), src, dst, et)
        s2r = s2.reshape(2, NP, H)
    return _kernel_f_final(hacc2, s2r, emh, erep, x)


_EREP = np.repeat(np.eye(H, dtype=np.float32), HD, axis=1)


# speedup vs baseline: 1.2276x; 1.0029x over previous
import functools

import numpy as np

import jax
import jax.numpy as jnp
from jax import lax
from jax.experimental import pallas as pl
from jax.experimental.pallas import tpu as pltpu
from jax.experimental.pallas import tpu_sc as plsc

N = 10000
E = 320000
D = 128
R = 8
H = 8
HD = 16

TN = 400
NT = N // TN
CH = 64
NCH = E // CH
NW = 32
KMAX = -(-NCH // NW)
NP = 10240
ROWS_PER_TILE = NP // 16
H2 = H // 2

F32 = jnp.float32
HIGH = jax.lax.Precision.HIGHEST


def _lrelu(t):
    return jnp.maximum(t, 0.2 * t)


def _finalize_h(hacc_ref, s_ref, emh_ref, erep_ref):
    ha = hacc_ref[0] + hacc_ref[1]
    s8 = (s_ref[0] + s_ref[1]) * emh_ref[...]
    denom = jnp.dot(s8, erep_ref[...], precision=HIGH) + 1e-10
    numer = ha * jnp.dot(emh_ref[...], erep_ref[...], precision=HIGH)
    return _lrelu(numer / denom)


def _bf16x3_dot(a, b):
    bf = jnp.bfloat16
    a_hi = a.astype(bf)
    a_lo = (a - a_hi.astype(F32)).astype(bf)
    b_hi = b.astype(bf)
    b_lo = (b - b_hi.astype(F32)).astype(bf)
    d = functools.partial(jnp.dot, preferred_element_type=F32)
    return d(a_hi, b_hi) + (d(a_hi, b_lo) + d(a_lo, b_hi))


def _a_compute(hb, wa_ref, z_ref, el_ref, er_ref, gelp_ref):
    t = _bf16x3_dot(hb, wa_ref[0])
    z_ref[0] = t[:, :D]
    elb = t[:, D:D + H]
    el_ref[0] = elb
    er_ref[0] = t[:, D + H:]
    gelp_ref[0, 0] = jnp.max(elb, axis=0, keepdims=True)


def _a0_body(h_ref, wa_ref, z_ref, el_ref, er_ref, gelp_ref):
    _a_compute(h_ref[...], wa_ref, z_ref, el_ref, er_ref, gelp_ref)


def _af_body(hacc_ref, s_ref, emh_ref, erep_ref, wa_ref,
             z_ref, el_ref, er_ref, gelp_ref, hb_ref):
    @pl.when(pl.program_id(1) == 0)
    def _():
        hb_ref[...] = _finalize_h(hacc_ref, s_ref, emh_ref, erep_ref)
    _a_compute(hb_ref[...], wa_ref, z_ref, el_ref, er_ref, gelp_ref)


NH128 = N * H // 128
TNH128 = TN * H // 128

_A_OUT_SPECS = [
    pl.BlockSpec((1, TN, D), lambda n, r: (r, n, 0)),
    pl.BlockSpec((1, TN, H), lambda n, r: (r, n, 0)),
    pl.BlockSpec((1, TN, H), lambda n, r: (r, n, 0)),
    pl.BlockSpec((1, 1, 1, H), lambda n, r: (n, r, 0, 0)),
]
_A_OUT_SHAPE = [
    jax.ShapeDtypeStruct((R, N, D), F32),
    jax.ShapeDtypeStruct((R, N, H), F32),
    jax.ShapeDtypeStruct((R, N, H), F32),
    jax.ShapeDtypeStruct((NT, R, 1, H), F32),
]
_A_W_SPECS = [
    pl.BlockSpec((1, D, D + 2 * H), lambda n, r: (r, 0, 0)),
]

_kernel_a0 = pl.pallas_call(
    _a0_body,
    grid=(NT, R),
    in_specs=[pl.BlockSpec((TN, D), lambda n, r: (n, 0))] + _A_W_SPECS,
    out_specs=_A_OUT_SPECS,
    out_shape=_A_OUT_SHAPE,
)

_kernel_af = pl.pallas_call(
    _af_body,
    grid=(NT, R),
    in_specs=[
        pl.BlockSpec((2, TN, D), lambda n, r: (0, n, 0)),
        pl.BlockSpec((2, TN, H), lambda n, r: (0, n, 0)),
        pl.BlockSpec((TN, H), lambda n, r: (n, 0)),
        pl.BlockSpec((H, D), lambda n, r: (0, 0)),
    ] + _A_W_SPECS,
    out_specs=_A_OUT_SPECS,
    out_shape=_A_OUT_SHAPE,
    scratch_shapes=[pltpu.VMEM((TN, D), F32)],
)


def _a2_body(gelp_ref, er_ref, emh_ref):
    gel = jnp.max(gelp_ref[...], axis=(0, 2))
    er = er_ref[...]
    t = _lrelu(gel[:, None, :] + er)
    emh_ref[...] = jnp.exp(-jnp.max(t, axis=0))


_kernel_a2 = pl.pallas_call(
    _a2_body,
    grid=(NT,),
    in_specs=[
        pl.BlockSpec((NT, R, 1, H), lambda n: (0, 0, 0, 0)),
        pl.BlockSpec((R, TN, H), lambda n: (0, n, 0)),
    ],
    out_specs=pl.BlockSpec((TN, H), lambda n: (n, 0)),
    out_shape=jax.ShapeDtypeStruct((N, H), F32),
)


_sc_mesh = plsc.VectorSubcoreMesh(core_axis_name="c", subcore_axis_name="s")

_NG = 2
_NS = 3
_NSLOT = 6
_KU = -(-(KMAX + 2) // _NSLOT)

_GSET = [
    pltpu.VMEM((CH,), jnp.int32),
    pltpu.VMEM((CH,), jnp.int32),
    pltpu.VMEM((CH,), jnp.int32),
    pltpu.VMEM((H2, 2 * CH), jnp.int32),
    pltpu.VMEM((H2, 2 * CH), jnp.int32),
    pltpu.VMEM((H2, 2 * CH), F32),
    pltpu.VMEM((H2, 2 * CH), F32),
    pltpu.SemaphoreType.DMA,
]
_SSET = [
    pltpu.VMEM((CH,), jnp.int32),
    pltpu.VMEM((H2, 2 * CH), jnp.int32),
    pltpu.VMEM((H2, 2 * CH), F32),
    pltpu.VMEM((CH, D), F32),
    pltpu.SemaphoreType.DMA,
]


def _sc_scratch():
    return _GSET * _NG + _SSET * _NS + [
        pltpu.VMEM((CH * H,), F32),
        pltpu.VMEM_SHARED((NP * H,), F32),
        pltpu.VMEM_SHARED((NP, D), F32),
    ]


@functools.partial(
    pl.kernel,
    out_type=(
        jax.ShapeDtypeStruct((2, NP * H), F32),
        jax.ShapeDtypeStruct((2, NP, D), F32),
    ),
    mesh=_sc_mesh,
    scratch_types=_sc_scratch(),
)
def _edge_kernel(el_hbm, er_hbm, z_hbm, src_hbm, dst_hbm, et_hbm,
                 s_out, hacc_out, *scr):
    ng, ns = len(_GSET), len(_SSET)
    gsets = [scr[g * ng:(g + 1) * ng] for g in range(_NG)]
    base = _NG * ng
    ssets = [scr[base + s * ns:base + (s + 1) * ns] for s in range(_NS)]
    zb1, s_sp, hacc_sp = scr[base + _NS * ns:]
    cid = lax.axis_index("c")
    sid = lax.axis_index("s")
    wid = sid * 2 + cid
    zeros16 = jnp.zeros((16,), F32)

    zg0 = ssets[0][3]

    def _zero_zg(i, _):
        for j in range(D // 16):
            zg0[i, pl.ds(j * 16, 16)] = zeros16
        return 0
    lax.fori_loop(0, CH, _zero_zg, 0)

    def _zero_zb1(i, _):
        zb1[pl.ds(i * 16, 16)] = zeros16
        return 0
    lax.fori_loop(0, CH * H // 16, _zero_zb1, 0)

    row0 = sid * ROWS_PER_TILE
    zsem = ssets[0][4]
    zpairs = []
    for t in range(ROWS_PER_TILE // CH):
        zpairs.append((zg0, hacc_sp.at[pl.ds(row0 + t * CH, CH)]))
        zpairs.append((zb1, s_sp.at[pl.ds((row0 + t * CH) * H, CH * H)]))
    for src, dst in zpairs:
        pltpu.async_copy(src, dst, zsem)
    for src, dst in zpairs:
        pltpu.make_async_copy(src, dst, zsem).wait()
    plsc.subcore_barrier()

    def fire_gathers(g, s, chunk_id):
        srcb, etb, idxrs, ixs, ixd, elg, erg, gsem = gsets[g]
        dstb, ixm, wb, zg, ssem = ssets[s]
        base = chunk_id * CH
        pltpu.sync_copy(src_hbm.at[pl.ds(base, CH)], srcb)
        pltpu.sync_copy(dst_hbm.at[pl.ds(base, CH)], dstb)
        pltpu.sync_copy(et_hbm.at[pl.ds(base, CH)], etb)

        def _idx(i, _):
            sl = pl.ds(i * 16, 16)
            etN = etb[sl] * N
            rs = etN + srcb[sl]
            idxrs[sl] = rs
            rs8 = rs * H
            rd8 = (etN + dstb[sl]) * H
            dm8 = dstb[sl] * H
            for k in range(H2):
                sl0 = pl.ds(i * 16, 16)
                sl1 = pl.ds(CH + i * 16, 16)
                ixs[k, sl0] = rs8 + (2 * k)
                ixs[k, sl1] = rs8 + (2 * k + 1)
                ixd[k, sl0] = rd8 + (2 * k)
                ixd[k, sl1] = rd8 + (2 * k + 1)
                ixm[k, sl0] = dm8 + (2 * k)
                ixm[k, sl1] = dm8 + (2 * k + 1)
            return 0
        lax.fori_loop(0, CH // 16, _idx, 0)
        for src, dst in _gather_pairs(g, s):
            pltpu.async_copy(src, dst, gsem)

    def _gather_pairs(g, s):
        srcb, etb, idxrs, ixs, ixd, elg, erg, gsem = gsets[g]
        zg = ssets[s][3]
        pairs = [(z_hbm.at[idxrs], zg)]
        for k in range(H2):
            pairs.append((el_hbm.at[ixs.at[k]], elg.at[k]))
            pairs.append((er_hbm.at[ixd.at[k]], erg.at[k]))
        return pairs

    def drain_gathers(g, s):
        gsem = gsets[g][7]
        for src, dst in _gather_pairs(g, s):
            pltpu.make_async_copy(src, dst, gsem).wait()

    def _scatter_pairs(s):
        dstb, ixm, wb, zg, ssem = ssets[s]
        pairs = [(wb.at[k], s_sp.at[ixm.at[k]]) for k in range(H2)]
        pairs.append((zg, hacc_sp.at[dstb]))
        return pairs

    def fire_scatters(s):
        ssem = ssets[s][4]
        for src, dst in _scatter_pairs(s):
            pltpu.async_copy(src, dst, ssem, add=True)

    def drain_scatters(s):
        ssem = ssets[s][4]
        for src, dst in _scatter_pairs(s):
            pltpu.make_async_copy(src, dst, ssem).wait()

    def compute(g, s):
        srcb, etb, idxrs, ixs, ixd, elg, erg, gsem = gsets[g]
        dstb, ixm, wb, zg, ssem = ssets[s]

        for k in range(H2):
            for v in range(2 * CH // 16):
                sl = pl.ds(v * 16, 16)
                t = elg[k, sl] + erg[k, sl]
                wb[k, sl] = jnp.exp(_lrelu(t))

        def _scale(cv, _):
            for h in range(H):
                wv = wb[h // 2, pl.ds((h % 2) * CH + cv * 16, 16)]
                sl = pl.ds(h * 16, 16)
                for t in range(16):
                    whc = wv.at[jnp.full((16,), t, jnp.int32)].get(
                        mode='promise_in_bounds')
                    c = cv * 16 + t
                    zg[c, sl] = zg[c, sl] * whc
            return 0
        lax.fori_loop(0, CH // 16, _scale, 0)

    fire_gathers(0, 0, wid)

    def _group(k6, _):
        for j in range(_NSLOT):
            i = _NSLOT * k6 + j
            c_i = wid + NW * i
            c_ip1 = c_i + NW
            c_im2 = c_i - 2 * NW

            @pl.when((i >= 2) & (c_im2 < NCH))
            def _():
                drain_scatters((j + 1) % _NS)

            @pl.when(c_ip1 < NCH)
            def _():
                fire_gathers((j + 1) % _NG, (j + 1) % _NS, c_ip1)

            @pl.when(c_i < NCH)
            def _():
                drain_gathers(j % _NG, j % _NS)
                compute(j % _NG, j % _NS)
                fire_scatters(j % _NS)
        return 0
    lax.fori_loop(0, _KU, _group, 0)

    plsc.subcore_barrier()
    pltpu.sync_copy(s_sp.at[pl.ds(row0 * H, ROWS_PER_TILE * H)],
                    s_out.at[cid].at[pl.ds(row0 * H, ROWS_PER_TILE * H)])
    pltpu.sync_copy(hacc_sp.at[pl.ds(row0, ROWS_PER_TILE)],
                    hacc_out.at[cid].at[pl.ds(row0, ROWS_PER_TILE)])


def _f_body(hacc_ref, s_ref, emh_ref, erep_ref, x_ref, out_ref):
    out_ref[...] = _finalize_h(hacc_ref, s_ref, emh_ref, erep_ref) + x_ref[...]


_kernel_f_final = pl.pallas_call(
    _f_body,
    grid=(NT,),
    in_specs=[
        pl.BlockSpec((2, TN, D), lambda n: (0, n, 0)),
        pl.BlockSpec((2, TN, H), lambda n: (0, n, 0)),
        pl.BlockSpec((TN, H), lambda n: (n, 0)),
        pl.BlockSpec((H, D), lambda n: (0, 0)),
        pl.BlockSpec((TN, D), lambda n: (n, 0)),
    ],
    out_specs=pl.BlockSpec((TN, D), lambda n: (n, 0)),
    out_shape=jax.ShapeDtypeStruct((N, D), F32),
)


def _a_layout(a):
    idx = jnp.arange(D)
    head = idx // HD
    return jnp.zeros((D, H), F32).at[idx, head].set(a.reshape(-1))


def kernel(x, edge_index, edge_type, W0, al0, ar0, W1, al1, ar1,
           W2, al2, ar2, W3, al3, ar3):
    src = edge_index[0]
    dst = edge_index[1]
    et = edge_type
    params = [(W0, al0, ar0), (W1, al1, ar1), (W2, al2, ar2), (W3, al3, ar3)]

    erep = jnp.asarray(_EREP)
    out = None
    for l, (W, al, ar) in enumerate(params):
        WA = jnp.concatenate(
            [W, jnp.matmul(W, _a_layout(al), precision=HIGH),
             jnp.matmul(W, _a_layout(ar), precision=HIGH)], axis=-1)
        if l == 0:
            z, el, er, gelp = _kernel_a0(x, WA)
        else:
            z, el, er, gelp = _kernel_af(hacc2, s2r, emh, erep, WA)
        emh = _kernel_a2(gelp, er)
        s2, hacc2 = _edge_kernel(el.reshape(-1), er.reshape(-1),
                                 z.reshape(R * N, D), src, dst, et)
        s2r = s2.reshape(2, NP, H)
    return _kernel_f_final(hacc2, s2r, emh, erep, x)


_EREP = np.repeat(np.eye(H, dtype=np.float32), HD, axis=1)
